# Initial kernel scaffold; baseline (speedup 1.0000x reference)
import jax
import jax.numpy as jnp
from jax.experimental import pallas as pl


def kernel(features, idx, feat_memory, pred_memory):
    raise NotImplementedError("write your pallas kernel here")


# speedup vs baseline: 3.2077x; 3.2077x over previous
import jax
import jax.numpy as jnp
from jax.experimental import pallas as pl
from jax.experimental.pallas import tpu as pltpu
from jax.experimental.pallas import tpu_sc as plsc

BATCH = 1024
FEAT = 128
N = 100000
K = 5
CLS = 100

BLK = 2000
NBLK = N // BLK
NEG = -1e30
POSBIG = 1e30
IBIG = 1 << 30


def _topk_body(f_ref, idx_ref, fm_ref, out_ref, fn_ref, vals_ref, cols_ref):
    i = pl.program_id(0)

    @pl.when(i == 0)
    def _init():
        f = f_ref[...]
        norm = jnp.sqrt(jnp.sum(f * f, axis=1, keepdims=True))
        fn_ref[...] = f / jnp.maximum(norm, 1e-12)
        vals_ref[...] = jnp.full((BATCH, 8), NEG, jnp.float32)
        cols_ref[...] = jnp.zeros((BATCH, 8), jnp.int32)

    fn = fn_ref[...]
    fm = fm_ref[...]
    dis = jax.lax.dot_general(fn, fm, (((1,), (1,)), ((), ())),
                              preferred_element_type=jnp.float32)
    col = i * BLK + jax.lax.broadcasted_iota(jnp.int32, (BATCH, BLK), 1)
    valid = (col < N) & (col != idx_ref[...])
    dis = jnp.where(valid, dis, NEG)

    run_v = vals_ref[...]
    run_c = cols_ref[...]
    for _ in range(K):
        m = jnp.max(dis, axis=1, keepdims=True)
        pos = jnp.min(jnp.where(dis == m, col, IBIG), axis=1, keepdims=True)
        dis = jnp.where(col == pos, NEG, dis)
        sh_v = jnp.concatenate(
            [jnp.full((BATCH, 1), POSBIG, jnp.float32), run_v[:, :7]], axis=1)
        sh_c = jnp.concatenate(
            [jnp.zeros((BATCH, 1), jnp.int32), run_c[:, :7]], axis=1)
        keep = run_v >= m
        above = sh_v >= m
        run_v = jnp.where(keep, run_v, jnp.where(above, m, sh_v))
        run_c = jnp.where(keep, run_c, jnp.where(above, pos, sh_c))
    vals_ref[...] = run_v
    cols_ref[...] = run_c
    out_ref[...] = run_c


def _topk_indices(features, idx2d, feat_memory):
    return pl.pallas_call(
        _topk_body,
        grid=(NBLK,),
        in_specs=[
            pl.BlockSpec((BATCH, FEAT), lambda i: (0, 0)),
            pl.BlockSpec((BATCH, 1), lambda i: (0, 0)),
            pl.BlockSpec((BLK, FEAT), lambda i: (i, 0)),
        ],
        out_specs=pl.BlockSpec((BATCH, 8), lambda i: (0, 0)),
        out_shape=jax.ShapeDtypeStruct((BATCH, 8), jnp.int32),
        scratch_shapes=[
            pltpu.VMEM((BATCH, FEAT), jnp.float32),
            pltpu.VMEM((BATCH, 8), jnp.float32),
            pltpu.VMEM((BATCH, 8), jnp.int32),
        ],
        compiler_params=pltpu.CompilerParams(
            dimension_semantics=("arbitrary",)),
    )(features, idx2d, feat_memory)


GW = 128
CPAD = 128


def _sc_gather(pred_padded, flat_idx):
    mesh = plsc.VectorSubcoreMesh(core_axis_name="c", subcore_axis_name="s")

    @pl.kernel(out_type=jax.ShapeDtypeStruct((BATCH * K, CPAD), jnp.float32),
               mesh=mesh)
    def k(pred_hbm, i_hbm, o_hbm):
        def body(i_vmem, o_vmem):
            pltpu.sync_copy(pred_hbm.at[i_vmem.at[0]], o_vmem)

        pltpu.emit_pipeline(
            body,
            grid=(BATCH * K // GW,),
            in_specs=[pl.BlockSpec((1, GW), lambda i: (0, i))],
            out_specs=[pl.BlockSpec((GW, CPAD), lambda i: (i, 0))],
            core_axis_name=("c", "s"),
            dimension_semantics=(pltpu.PARALLEL,),
        )(i_hbm, o_hbm)

    return k(pred_padded, flat_idx)


def _mean_argmax_body(g_ref, mean_ref, lab_ref):
    g = g_ref[...]
    acc = g[:, 0:CLS]
    for k in range(1, K):
        acc = acc + g[:, k * CPAD:k * CPAD + CLS]
    mean = acc * (1.0 / K)
    mean_ref[...] = mean
    m = jnp.max(mean, axis=1, keepdims=True)
    ci = jax.lax.broadcasted_iota(jnp.int32, (BATCH, CLS), 1)
    lab_ref[...] = jnp.min(jnp.where(mean == m, ci, IBIG), axis=1,
                           keepdims=True)


def _mean_argmax(gathered2d):
    return pl.pallas_call(
        _mean_argmax_body,
        out_shape=[jax.ShapeDtypeStruct((BATCH, CLS), jnp.float32),
                   jax.ShapeDtypeStruct((BATCH, 1), jnp.int32)],
    )(gathered2d)


def kernel(features, idx, feat_memory, pred_memory):
    idx2d = idx.astype(jnp.int32).reshape(BATCH, 1)
    top = _topk_indices(features, idx2d, feat_memory)
    flat_idx = top[:, :K].reshape(1, BATCH * K)
    pred_padded = jnp.pad(pred_memory, ((0, 0), (0, CPAD - CLS)))
    gathered = _sc_gather(pred_padded, flat_idx)
    mean_logits, lab = _mean_argmax(gathered.reshape(BATCH, K * CPAD))
    return lab.reshape(BATCH), mean_logits
